# baseline (speedup 1.0000x reference)
import functools

import jax
import jax.numpy as jnp
from jax import lax
from jax.experimental import pallas as pl
from jax.experimental.pallas import tpu as pltpu
from jax.experimental.pallas import tpu_sc as plsc

V = 50000
K = 256
BH = 2048
NB = 25
VP = BH * NB
NW = 32
CH = VP // NW
CQ = 1664
NG = CH // 16
Q_MIN = 0.5
S_B = 1.0
_DN = (((1,), (0,)), ((), ()))


def _atanh(b):
    return 0.5 * jnp.log((1.0 + b) / (1.0 - b))


def _dot3(a, b16):
    hi = a.astype(jnp.bfloat16)
    r1 = a - hi.astype(jnp.float32)
    mid = r1.astype(jnp.bfloat16)
    lo = (r1 - mid.astype(jnp.float32)).astype(jnp.bfloat16)
    o = jax.lax.dot_general(hi, b16, _DN, preferred_element_type=jnp.float32)
    o = o + jax.lax.dot_general(mid, b16, _DN,
                                preferred_element_type=jnp.float32)
    o = o + jax.lax.dot_general(lo, b16, _DN,
                                preferred_element_type=jnp.float32)
    return o


def _stage_a(featT_ref, sidxC_ref, sidxR_ref, betaC_ref,
             hit_ref, maxb_ref, misc_ref, maxb_s):
    b = pl.program_id(0)

    @pl.when(b == 0)
    def _init():
        maxb_s[...] = jnp.full((8, K), -1.0, jnp.float32)
        misc_ref[...] = jnp.zeros((8, BH), jnp.float32)

    s = sidxC_ref[...]
    hid = b * BH + jax.lax.broadcasted_iota(jnp.int32, (BH, 1), 0)
    nn_c = jnp.logical_and(s >= 0, hid < V)
    sc = jnp.clip(s, 0, K - 1)
    kiota = jax.lax.broadcasted_iota(jnp.int32, (BH, K), 1)
    onehot = jnp.logical_and(sc == kiota, nn_c)
    beta_c = jnp.clip(betaC_ref[...], 1e-4, 1.0 - 1e-4)
    cand = jnp.where(onehot, beta_c, -1.0)
    bm8 = jnp.max(cand.reshape(BH // 8, 8, K), axis=0)
    maxb_s[...] = jnp.maximum(maxb_s[...], bm8)

    sr = sidxR_ref[0]
    hid_r = b * BH + jax.lax.broadcasted_iota(jnp.int32, (1, BH), 1)
    vm = jnp.logical_and(hid_r >= 0, hid_r < V).astype(jnp.float32)
    nn = (jnp.logical_and(sr >= 0, hid_r < V)).astype(jnp.float32)
    beta = jnp.clip(featT_ref[0:1, :], 1e-4, 1.0 - 1e-4)
    x0 = featT_ref[1:2, :]
    x1 = featT_ref[2:3, :]
    pe = featT_ref[3:4, :]
    te = featT_ref[4:5, :]
    pp0 = featT_ref[5:6, :]
    pp1 = featT_ref[6:7, :]
    tp0 = featT_ref[7:8, :]
    tp1 = featT_ref[8:9, :]
    tt = featT_ref[9:10, :]
    q = (_atanh(beta) ** 2 + Q_MIN) * tt
    pad1 = 1.0 - vm
    el = (te - pe) ** 2 / (te * te + pad1)
    pll = ((tp0 - pp0) ** 2 / (tp0 * tp0 + pad1)
           + (tp1 - pp1) ** 2 / (tp1 * tp1 + pad1))
    pw = beta * nn
    hit_ref[...] = jnp.concatenate(
        [beta, x0, x1, q, pw, pw * el, pw * pll, nn], axis=0)

    isn = vm * (1.0 - nn)
    idv = featT_ref[10:16, :]
    idsq = jnp.sum(idv * idv, axis=0, keepdims=True)
    misc_ref[...] = misc_ref[...] + jnp.concatenate(
        [beta * isn, isn, idsq, jnp.zeros((5, BH), jnp.float32)], axis=0)

    @pl.when(b == NB - 1)
    def _finish():
        m = jnp.max(maxb_s[...], axis=0, keepdims=True)
        maxb_ref[...] = jnp.broadcast_to(m, (8, K))


@jax.jit
def _run_a(featT_p, sidxC_p, sidxR_p, betaC_p):
    return pl.pallas_call(
        _stage_a,
        grid=(NB,),
        in_specs=[
            pl.BlockSpec((16, BH), lambda b: (0, b)),
            pl.BlockSpec((BH, 1), lambda b: (b, 0)),
            pl.BlockSpec((1, 1, BH), lambda b: (b, 0, 0)),
            pl.BlockSpec((BH, 1), lambda b: (b, 0)),
        ],
        out_specs=[
            pl.BlockSpec((8, BH), lambda b: (0, b)),
            pl.BlockSpec((8, K), lambda b: (0, 0)),
            pl.BlockSpec((8, BH), lambda b: (0, 0)),
        ],
        out_shape=[
            jax.ShapeDtypeStruct((8, VP), jnp.float32),
            jax.ShapeDtypeStruct((8, K), jnp.float32),
            jax.ShapeDtypeStruct((8, BH), jnp.float32),
        ],
        scratch_shapes=[pltpu.VMEM((8, K), jnp.float32)],
    )(featT_p, sidxC_p, sidxR_p, betaC_p)


def _stage_b(b_hbm, x0_hbm, x1_hbm, q_hbm, pw_hbm, pe_hbm, pp_hbm,
             sidx_hbm, maxb_hbm, out_hbm,
             idx_v, b_v, x0_v, x1_v, q_v, pw_v, pe_v, pp_v,
             scl_v, mb_v, val_f, ix_f, acc_v, acc_sh, sem):
    wid = lax.axis_index("s") * 2 + lax.axis_index("c")
    base = wid * CH
    pltpu.sync_copy(sidx_hbm.at[pl.ds(base, CH)], idx_v)
    pltpu.sync_copy(b_hbm.at[pl.ds(base, CH)], b_v)
    pltpu.sync_copy(x0_hbm.at[pl.ds(base, CH)], x0_v)
    pltpu.sync_copy(x1_hbm.at[pl.ds(base, CH)], x1_v)
    pltpu.sync_copy(q_hbm.at[pl.ds(base, CH)], q_v)
    pltpu.sync_copy(pw_hbm.at[pl.ds(base, CH)], pw_v)
    pltpu.sync_copy(pe_hbm.at[pl.ds(base, CH)], pe_v)
    pltpu.sync_copy(pp_hbm.at[pl.ds(base, CH)], pp_v)

    sid = lax.axis_index("s")

    @pl.when(sid == 0)
    def _zero_shared():
        def zero(i, _):
            acc_v[pl.ds(i * 16, 16)] = jnp.zeros((16,), jnp.float32)
            return _
        lax.fori_loop(0, (9 * K) // 16, zero, 0)
        pltpu.sync_copy(acc_v, acc_sh)

    def clipidx(g, _):
        o = g * 16
        scl_v[pl.ds(o, 16)] = jnp.clip(idx_v[pl.ds(o, 16)], 0, K - 1)
        return _

    lax.fori_loop(0, NG, clipidx, 0)
    chunks = [(c * 128, 128) for c in range(CH // 128)]
    if CH % 128:
        chunks.append(((CH // 128) * 128, CH % 128))
    cps = [pltpu.async_copy(maxb_hbm.at[scl_v.at[pl.ds(o, n)]],
                            mb_v.at[pl.ds(o, n)], sem)
           for o, n in chunks]
    for cp in cps:
        cp.wait()

    def body(g, _):
        o = g * 16
        idx = idx_v[pl.ds(o, 16)]
        nnb = idx >= 0
        nnf = jnp.where(nnb, 1.0, 0.0)
        scl = jnp.clip(idx, 0, K - 1)
        mb = mb_v[pl.ds(o, 16)]
        beta = b_v[pl.ds(o, 16)]
        is_a = jnp.where(jnp.logical_and(beta == mb, nnb), 1.0, 0.0)
        vals = (nnf, is_a, is_a * x0_v[pl.ds(o, 16)],
                is_a * x1_v[pl.ds(o, 16)], is_a * q_v[pl.ds(o, 16)],
                is_a * beta, pw_v[pl.ds(o, 16)], pe_v[pl.ds(o, 16)],
                pp_v[pl.ds(o, 16)])
        for j, v in enumerate(vals):
            val_f[pl.ds(j * CQ + o, 16)] = v
            ix_f[pl.ds(j * CQ + o, 16)] = scl + j * K
        return _

    lax.fori_loop(0, NG, body, 0)
    for j in range(9):
        for t in range(CH, CQ, 16):
            val_f[pl.ds(j * CQ + t, 16)] = jnp.zeros((16,), jnp.float32)
            ix_f[pl.ds(j * CQ + t, 16)] = jnp.zeros((16,), jnp.int32)
    plsc.subcore_barrier()
    pltpu.sync_copy(val_f, acc_sh.at[ix_f], add=True)
    plsc.subcore_barrier()

    @pl.when(sid == 0)
    def _writeback():
        cid = lax.axis_index("c")
        pltpu.sync_copy(acc_sh, out_hbm.at[pl.ds(cid * 9 * K, 9 * K)])


@jax.jit
def _run_b(hit_p, sidx1d, maxb8):
    rows = [hit_p[i] for i in range(7)]
    maxb_row = maxb8[0]
    mesh = plsc.VectorSubcoreMesh(core_axis_name="c", subcore_axis_name="s",
                                  num_cores=2, num_subcores=16)
    f = pl.kernel(
        _stage_b,
        out_type=jax.ShapeDtypeStruct((2 * 9 * K,), jnp.float32),
        mesh=mesh,
        scratch_types=[
            pltpu.VMEM((CH,), jnp.int32),
            pltpu.VMEM((CH,), jnp.float32),
            pltpu.VMEM((CH,), jnp.float32),
            pltpu.VMEM((CH,), jnp.float32),
            pltpu.VMEM((CH,), jnp.float32),
            pltpu.VMEM((CH,), jnp.float32),
            pltpu.VMEM((CH,), jnp.float32),
            pltpu.VMEM((CH,), jnp.float32),
            pltpu.VMEM((CH,), jnp.int32),
            pltpu.VMEM((CH,), jnp.float32),
            pltpu.VMEM((9 * CQ,), jnp.float32),
            pltpu.VMEM((9 * CQ,), jnp.int32),
            pltpu.VMEM((9 * K,), jnp.float32),
            pltpu.VMEM_SHARED((9 * K,), jnp.float32),
            pltpu.SemaphoreType.DMA,
        ],
    )
    out = f(*rows, sidx1d, maxb_row)
    return jnp.reshape(out, (2, 9 * K))


def _stage_c(hit_ref, sidxR_ref, part_ref, misc_ref, out_ref,
             alpha_s, alphaT_s, repacc_s, sacc_s, scal_s):
    b = pl.program_id(0)

    @pl.when(b == 0)
    def _epilogue():
        repacc_s[...] = jnp.zeros((8, BH), jnp.float32)
        sacc_s[...] = jnp.zeros((8, BH), jnp.float32)
        sums = jnp.sum(part_ref[...].reshape(2, 9, K), axis=0)
        nh = sums[0:1, :]
        den = sums[1:2, :] + 1e-9
        xa0 = sums[2:3, :] / den
        xa1 = sums[3:4, :] / den
        qa = sums[4:5, :] / den
        ba = sums[5:6, :] / den
        exists = (nh > 0.0).astype(jnp.float32)
        plden = sums[6:7, :] + 1e-9
        pl0 = sums[7:8, :] / plden
        pl1 = sums[8:9, :] / plden
        wk = qa * exists
        arows = jnp.concatenate(
            [xa0, xa1, qa, wk, jnp.zeros((4, K), jnp.float32)], axis=0)
        alpha_s[...] = arows
        alphaT_s[...] = jax.lax.transpose(arows, (1, 0))
        n_obj = jnp.sum(exists) + 1e-9
        minb = jnp.sum((1.0 - ba) * exists) / n_obj
        payload = jnp.sum((pl0 + pl1) * exists) / n_obj
        noise_num = jnp.sum(misc_ref[0:1, :])
        noise_den = jnp.sum(misc_ref[1:2, :])
        idsq = jnp.sum(misc_ref[2:3, :])
        noise = S_B * noise_num / (noise_den + 1e-9)
        cls = 1e-8 * idsq / (V * 6.0)
        scal_s[5] = minb + payload + noise + cls

    sr = sidxR_ref[0]
    sc = jnp.clip(sr, 0, K - 1)
    kiota = jax.lax.broadcasted_iota(jnp.int32, (K, BH), 0)
    onehotT16 = jnp.logical_and(sc == kiota, sr >= 0).astype(jnp.bfloat16)
    x0 = hit_ref[1:2, :]
    x1 = hit_ref[2:3, :]
    q = hit_ref[3:4, :]
    nn = hit_ref[7:8, :]
    qb = q * nn
    gT = _dot3(alpha_s[...], onehotT16)
    xa0h = gT[0:1, :]
    xa1h = gT[1:2, :]
    qah = gT[2:3, :]
    wkh = gT[3:4, :]
    d2a = (x0 - xa0h) ** 2 + (x1 - xa1h) ** 2
    att_r = (qb * qah) * d2a
    hs = jnp.maximum(0.0, 1.0 - jnp.sqrt(d2a + 1e-9))
    same_r = (hs * qb) * wkh
    xa0c = alphaT_s[:, 0:1]
    xa1c = alphaT_s[:, 1:2]
    wkc = alphaT_s[:, 3:4]
    d2 = (x0 - xa0c) ** 2 + (x1 - xa1c) ** 2
    hinge = jnp.maximum(0.0, 1.0 - jnp.sqrt(d2 + 1e-9))
    repm = (hinge * qb) * wkc
    repacc_s[...] = repacc_s[...] + jnp.sum(repm.reshape(K // 8, 8, BH), axis=0)
    sacc_s[...] = sacc_s[...] + jnp.concatenate(
        [att_r, same_r, jnp.zeros((6, BH), jnp.float32)], axis=0)

    @pl.when(b == NB - 1)
    def _final():
        att = jnp.sum(sacc_s[0:1, :])
        corr = jnp.sum(sacc_s[1:2, :])
        rep = jnp.sum(repacc_s[...]) - corr
        loss = att / V + rep / V + scal_s[5]
        out_ref[...] = jnp.reshape(loss, (1, 1))


@jax.jit
def _run_c(hit_p, sidxR_p, part, misc):
    return pl.pallas_call(
        _stage_c,
        grid=(NB,),
        in_specs=[
            pl.BlockSpec((8, BH), lambda b: (0, b)),
            pl.BlockSpec((1, 1, BH), lambda b: (b, 0, 0)),
            pl.BlockSpec((2, 9 * K), lambda b: (0, 0)),
            pl.BlockSpec((8, BH), lambda b: (0, 0)),
        ],
        out_specs=pl.BlockSpec((1, 1), lambda b: (0, 0)),
        out_shape=jax.ShapeDtypeStruct((1, 1), jnp.float32),
        scratch_shapes=[
            pltpu.VMEM((8, K), jnp.float32),
            pltpu.VMEM((K, 8), jnp.float32),
            pltpu.VMEM((8, BH), jnp.float32),
            pltpu.VMEM((8, BH), jnp.float32),
            pltpu.SMEM((8,), jnp.float32),
        ],
    )(hit_p, sidxR_p, part, misc)


def kernel(pred_beta, pred_ccoords, pred_energy, pred_pos, pred_time,
           pred_id, t_idx, t_energy, t_pos, t_time, t_pid, rowsplits):
    feat = jnp.concatenate(
        [pred_beta, pred_ccoords, pred_energy, t_energy, pred_pos, t_pos,
         t_time, pred_id], axis=1)
    pad = VP - V
    featT_p = jnp.pad(feat.T, ((0, 0), (0, pad)))
    sidx_p = jnp.pad(t_idx, ((0, pad), (0, 0)), constant_values=-1)
    sidxR_p = jnp.reshape(sidx_p, (NB, 1, BH))
    betaC_p = jnp.pad(pred_beta, ((0, pad), (0, 0)))
    hit_p, maxb8, misc = _run_a(featT_p, sidx_p, sidxR_p, betaC_p)
    part = _run_b(hit_p, jnp.reshape(sidx_p, (VP,)), maxb8)
    loss = _run_c(hit_p, sidxR_p, part, misc)
    return pred_beta, jnp.reshape(loss, (1,))


# speedup vs baseline: 1.0154x; 1.0154x over previous
import functools

import jax
import jax.numpy as jnp
from jax import lax
from jax.experimental import pallas as pl
from jax.experimental.pallas import tpu as pltpu
from jax.experimental.pallas import tpu_sc as plsc

V = 50000
K = 256
BH = 2048
NB = 25
VP = BH * NB
NW = 32
CH = VP // NW
CQ = 1664
NG = CH // 16
NQ = 7
Q_MIN = 0.5
S_B = 1.0
_DN = (((1,), (0,)), ((), ()))


def _atanh(b):
    return 0.5 * jnp.log((1.0 + b) / (1.0 - b))


def _dot3(a, b16):
    hi = a.astype(jnp.bfloat16)
    r1 = a - hi.astype(jnp.float32)
    mid = r1.astype(jnp.bfloat16)
    lo = (r1 - mid.astype(jnp.float32)).astype(jnp.bfloat16)
    o = jax.lax.dot_general(hi, b16, _DN, preferred_element_type=jnp.float32)
    o = o + jax.lax.dot_general(mid, b16, _DN,
                                preferred_element_type=jnp.float32)
    o = o + jax.lax.dot_general(lo, b16, _DN,
                                preferred_element_type=jnp.float32)
    return o


def _stage_a(featT_ref, sidxC_ref, sidxR_ref, betaC_ref,
             hit_ref, maxb_ref, misc_ref, maxb_s):
    b = pl.program_id(0)

    @pl.when(b == 0)
    def _init():
        maxb_s[...] = jnp.full((8, K), -1.0, jnp.float32)
        misc_ref[...] = jnp.zeros((8, BH), jnp.float32)

    s = sidxC_ref[...]
    hid = b * BH + jax.lax.broadcasted_iota(jnp.int32, (BH, 1), 0)
    nn_c = jnp.logical_and(s >= 0, hid < V)
    sc = jnp.clip(s, 0, K - 1)
    kiota = jax.lax.broadcasted_iota(jnp.int32, (BH, K), 1)
    onehot = jnp.logical_and(sc == kiota, nn_c)
    beta_c = jnp.clip(betaC_ref[...], 1e-4, 1.0 - 1e-4)
    cand = jnp.where(onehot, beta_c, -1.0)
    bm8 = jnp.max(cand.reshape(BH // 8, 8, K), axis=0)
    maxb_s[...] = jnp.maximum(maxb_s[...], bm8)

    sr = sidxR_ref[0]
    hid_r = b * BH + jax.lax.broadcasted_iota(jnp.int32, (1, BH), 1)
    vm = jnp.logical_and(hid_r >= 0, hid_r < V).astype(jnp.float32)
    nn = (jnp.logical_and(sr >= 0, hid_r < V)).astype(jnp.float32)
    beta = jnp.clip(featT_ref[0:1, :], 1e-4, 1.0 - 1e-4)
    x0 = featT_ref[1:2, :]
    x1 = featT_ref[2:3, :]
    pe = featT_ref[3:4, :]
    te = featT_ref[4:5, :]
    pp0 = featT_ref[5:6, :]
    pp1 = featT_ref[6:7, :]
    tp0 = featT_ref[7:8, :]
    tp1 = featT_ref[8:9, :]
    tt = featT_ref[9:10, :]
    q = (_atanh(beta) ** 2 + Q_MIN) * tt
    pad1 = 1.0 - vm
    el = (te - pe) ** 2 / (te * te + pad1)
    pll = ((tp0 - pp0) ** 2 / (tp0 * tp0 + pad1)
           + (tp1 - pp1) ** 2 / (tp1 * tp1 + pad1))
    pw = beta * nn
    hit_ref[...] = jnp.concatenate(
        [beta, x0, x1, q, pw, pw * el, pw * pll, nn], axis=0)

    isn = vm * (1.0 - nn)
    idv = featT_ref[10:16, :]
    idsq = jnp.sum(idv * idv, axis=0, keepdims=True)
    misc_ref[...] = misc_ref[...] + jnp.concatenate(
        [beta * isn, isn, idsq, jnp.zeros((5, BH), jnp.float32)], axis=0)

    @pl.when(b == NB - 1)
    def _finish():
        m = jnp.max(maxb_s[...], axis=0, keepdims=True)
        maxb_ref[...] = jnp.broadcast_to(m, (8, K))


@jax.jit
def _run_a(featT_p, sidxC_p, sidxR_p, betaC_p):
    return pl.pallas_call(
        _stage_a,
        grid=(NB,),
        in_specs=[
            pl.BlockSpec((16, BH), lambda b: (0, b)),
            pl.BlockSpec((BH, 1), lambda b: (b, 0)),
            pl.BlockSpec((1, 1, BH), lambda b: (b, 0, 0)),
            pl.BlockSpec((BH, 1), lambda b: (b, 0)),
        ],
        out_specs=[
            pl.BlockSpec((8, BH), lambda b: (0, b)),
            pl.BlockSpec((8, K), lambda b: (0, 0)),
            pl.BlockSpec((8, BH), lambda b: (0, 0)),
        ],
        out_shape=[
            jax.ShapeDtypeStruct((8, VP), jnp.float32),
            jax.ShapeDtypeStruct((8, K), jnp.float32),
            jax.ShapeDtypeStruct((8, BH), jnp.float32),
        ],
        scratch_shapes=[pltpu.VMEM((8, K), jnp.float32)],
    )(featT_p, sidxC_p, sidxR_p, betaC_p)


def _stage_b(b_hbm, x0_hbm, x1_hbm, q_hbm, pw_hbm, pe_hbm, pp_hbm,
             sidx_hbm, maxb_hbm, out_hbm,
             idx_v, b_v, x0_v, x1_v, q_v, pw_v, pe_v, pp_v,
             scl_v, mb_v, val_f, ix_f, acc_v, acc_sh, sem):
    wid = lax.axis_index("s") * 2 + lax.axis_index("c")
    base = wid * CH
    pltpu.sync_copy(sidx_hbm.at[pl.ds(base, CH)], idx_v)
    pltpu.sync_copy(b_hbm.at[pl.ds(base, CH)], b_v)
    pltpu.sync_copy(x0_hbm.at[pl.ds(base, CH)], x0_v)
    pltpu.sync_copy(x1_hbm.at[pl.ds(base, CH)], x1_v)
    pltpu.sync_copy(q_hbm.at[pl.ds(base, CH)], q_v)
    pltpu.sync_copy(pw_hbm.at[pl.ds(base, CH)], pw_v)
    pltpu.sync_copy(pe_hbm.at[pl.ds(base, CH)], pe_v)
    pltpu.sync_copy(pp_hbm.at[pl.ds(base, CH)], pp_v)

    sid = lax.axis_index("s")

    @pl.when(sid == 0)
    def _zero_shared():
        def zero(i, _):
            acc_v[pl.ds(i * 16, 16)] = jnp.zeros((16,), jnp.float32)
            return _
        lax.fori_loop(0, (4 * NQ * K) // 16, zero, 0)
        pltpu.sync_copy(acc_v, acc_sh)

    def clipidx(g, _):
        o = g * 16
        scl_v[pl.ds(o, 16)] = jnp.clip(idx_v[pl.ds(o, 16)], 0, K - 1)
        return _

    lax.fori_loop(0, NG, clipidx, 0)
    chunks = [(c * 128, 128) for c in range(CH // 128)]
    if CH % 128:
        chunks.append(((CH // 128) * 128, CH % 128))
    cps = [pltpu.async_copy(maxb_hbm.at[scl_v.at[pl.ds(o, n)]],
                            mb_v.at[pl.ds(o, n)], sem)
           for o, n in chunks]
    for cp in cps:
        cp.wait()

    bank = sid % 4

    def body(g, _):
        o = g * 16
        idx = idx_v[pl.ds(o, 16)]
        nnb = idx >= 0
        scl = jnp.clip(idx, 0, K - 1)
        mb = mb_v[pl.ds(o, 16)]
        beta = b_v[pl.ds(o, 16)]
        is_a = jnp.where(jnp.logical_and(beta == mb, nnb), 1.0, 0.0)
        vals = (is_a, is_a * x0_v[pl.ds(o, 16)],
                is_a * x1_v[pl.ds(o, 16)], is_a * q_v[pl.ds(o, 16)],
                pw_v[pl.ds(o, 16)], pe_v[pl.ds(o, 16)],
                pp_v[pl.ds(o, 16)])
        for j, v in enumerate(vals):
            val_f[pl.ds(j * CQ + o, 16)] = v
            ix_f[pl.ds(j * CQ + o, 16)] = scl + (bank * NQ + j) * K
        return _

    lax.fori_loop(0, NG, body, 0)
    for j in range(NQ):
        for t in range(CH, CQ, 16):
            val_f[pl.ds(j * CQ + t, 16)] = jnp.zeros((16,), jnp.float32)
            ix_f[pl.ds(j * CQ + t, 16)] = jnp.zeros((16,), jnp.int32)
    plsc.subcore_barrier()
    pltpu.sync_copy(val_f, acc_sh.at[ix_f], add=True)
    plsc.subcore_barrier()

    @pl.when(sid == 0)
    def _writeback():
        cid = lax.axis_index("c")
        pltpu.sync_copy(acc_sh, out_hbm.at[pl.ds(cid * 4 * NQ * K, 4 * NQ * K)])


@jax.jit
def _run_b(hit_p, sidx1d, maxb8):
    rows = [hit_p[i] for i in range(7)]
    maxb_row = maxb8[0]
    mesh = plsc.VectorSubcoreMesh(core_axis_name="c", subcore_axis_name="s",
                                  num_cores=2, num_subcores=16)
    f = pl.kernel(
        _stage_b,
        out_type=jax.ShapeDtypeStruct((8 * NQ * K,), jnp.float32),
        mesh=mesh,
        scratch_types=[
            pltpu.VMEM((CH,), jnp.int32),
            pltpu.VMEM((CH,), jnp.float32),
            pltpu.VMEM((CH,), jnp.float32),
            pltpu.VMEM((CH,), jnp.float32),
            pltpu.VMEM((CH,), jnp.float32),
            pltpu.VMEM((CH,), jnp.float32),
            pltpu.VMEM((CH,), jnp.float32),
            pltpu.VMEM((CH,), jnp.float32),
            pltpu.VMEM((CH,), jnp.int32),
            pltpu.VMEM((CH,), jnp.float32),
            pltpu.VMEM((NQ * CQ,), jnp.float32),
            pltpu.VMEM((NQ * CQ,), jnp.int32),
            pltpu.VMEM((4 * NQ * K,), jnp.float32),
            pltpu.VMEM_SHARED((4 * NQ * K,), jnp.float32),
            pltpu.SemaphoreType.DMA,
        ],
    )
    out = f(*rows, sidx1d, maxb_row)
    return jnp.reshape(out, (8, NQ * K))


def _stage_c(hit_ref, sidxR_ref, part_ref, misc_ref, maxb_ref, out_ref,
             alpha_s, alphaT_s, repacc_s, sacc_s, scal_s):
    b = pl.program_id(0)

    @pl.when(b == 0)
    def _epilogue():
        repacc_s[...] = jnp.zeros((8, BH), jnp.float32)
        sacc_s[...] = jnp.zeros((8, BH), jnp.float32)
        sums = jnp.sum(part_ref[...].reshape(8, NQ, K), axis=0)
        den = sums[0:1, :] + 1e-9
        xa0 = sums[1:2, :] / den
        xa1 = sums[2:3, :] / den
        qa = sums[3:4, :] / den
        ba = maxb_ref[0:1, :]
        exists = (sums[0:1, :] > 0.0).astype(jnp.float32)
        plden = sums[4:5, :] + 1e-9
        pl0 = sums[5:6, :] / plden
        pl1 = sums[6:7, :] / plden
        wk = qa * exists
        arows = jnp.concatenate(
            [xa0, xa1, qa, wk, jnp.zeros((4, K), jnp.float32)], axis=0)
        alpha_s[...] = arows
        alphaT_s[...] = jax.lax.transpose(arows, (1, 0))
        n_obj = jnp.sum(exists) + 1e-9
        minb = jnp.sum((1.0 - ba) * exists) / n_obj
        payload = jnp.sum((pl0 + pl1) * exists) / n_obj
        noise_num = jnp.sum(misc_ref[0:1, :])
        noise_den = jnp.sum(misc_ref[1:2, :])
        idsq = jnp.sum(misc_ref[2:3, :])
        noise = S_B * noise_num / (noise_den + 1e-9)
        cls = 1e-8 * idsq / (V * 6.0)
        scal_s[5] = minb + payload + noise + cls

    sr = sidxR_ref[0]
    sc = jnp.clip(sr, 0, K - 1)
    kiota = jax.lax.broadcasted_iota(jnp.int32, (K, BH), 0)
    onehotT16 = jnp.logical_and(sc == kiota, sr >= 0).astype(jnp.bfloat16)
    x0 = hit_ref[1:2, :]
    x1 = hit_ref[2:3, :]
    q = hit_ref[3:4, :]
    nn = hit_ref[7:8, :]
    qb = q * nn
    gT = _dot3(alpha_s[...], onehotT16)
    xa0h = gT[0:1, :]
    xa1h = gT[1:2, :]
    qah = gT[2:3, :]
    wkh = gT[3:4, :]
    d2a = (x0 - xa0h) ** 2 + (x1 - xa1h) ** 2
    att_r = (qb * qah) * d2a
    hs = jnp.maximum(0.0, 1.0 - jnp.sqrt(d2a + 1e-9))
    same_r = (hs * qb) * wkh
    xa0c = alphaT_s[:, 0:1]
    xa1c = alphaT_s[:, 1:2]
    wkc = alphaT_s[:, 3:4]
    d2 = (x0 - xa0c) ** 2 + (x1 - xa1c) ** 2
    hinge = jnp.maximum(0.0, 1.0 - jnp.sqrt(d2 + 1e-9))
    repm = (hinge * qb) * wkc
    repacc_s[...] = repacc_s[...] + jnp.sum(repm.reshape(K // 8, 8, BH), axis=0)
    sacc_s[...] = sacc_s[...] + jnp.concatenate(
        [att_r, same_r, jnp.zeros((6, BH), jnp.float32)], axis=0)

    @pl.when(b == NB - 1)
    def _final():
        att = jnp.sum(sacc_s[0:1, :])
        corr = jnp.sum(sacc_s[1:2, :])
        rep = jnp.sum(repacc_s[...]) - corr
        loss = att / V + rep / V + scal_s[5]
        out_ref[...] = jnp.reshape(loss, (1, 1))


@jax.jit
def _run_c(hit_p, sidxR_p, part, misc, maxb8):
    return pl.pallas_call(
        _stage_c,
        grid=(NB,),
        in_specs=[
            pl.BlockSpec((8, BH), lambda b: (0, b)),
            pl.BlockSpec((1, 1, BH), lambda b: (b, 0, 0)),
            pl.BlockSpec((8, NQ * K), lambda b: (0, 0)),
            pl.BlockSpec((8, BH), lambda b: (0, 0)),
            pl.BlockSpec((8, K), lambda b: (0, 0)),
        ],
        out_specs=pl.BlockSpec((1, 1), lambda b: (0, 0)),
        out_shape=jax.ShapeDtypeStruct((1, 1), jnp.float32),
        scratch_shapes=[
            pltpu.VMEM((8, K), jnp.float32),
            pltpu.VMEM((K, 8), jnp.float32),
            pltpu.VMEM((8, BH), jnp.float32),
            pltpu.VMEM((8, BH), jnp.float32),
            pltpu.SMEM((8,), jnp.float32),
        ],
    )(hit_p, sidxR_p, part, misc, maxb8)


def kernel(pred_beta, pred_ccoords, pred_energy, pred_pos, pred_time,
           pred_id, t_idx, t_energy, t_pos, t_time, t_pid, rowsplits):
    feat = jnp.concatenate(
        [pred_beta, pred_ccoords, pred_energy, t_energy, pred_pos, t_pos,
         t_time, pred_id], axis=1)
    pad = VP - V
    featT_p = jnp.pad(feat.T, ((0, 0), (0, pad)))
    sidx_p = jnp.pad(t_idx, ((0, pad), (0, 0)), constant_values=-1)
    sidxR_p = jnp.reshape(sidx_p, (NB, 1, BH))
    betaC_p = jnp.pad(pred_beta, ((0, pad), (0, 0)))
    hit_p, maxb8, misc = _run_a(featT_p, sidx_p, sidxR_p, betaC_p)
    part = _run_b(hit_p, jnp.reshape(sidx_p, (VP,)), maxb8)
    loss = _run_c(hit_p, sidxR_p, part, misc, maxb8)
    return pred_beta, jnp.reshape(loss, (1,))
